# baseline (speedup 1.0000x reference)
import jax
import jax.numpy as jnp
from jax.experimental import pallas as pl
from jax.experimental.pallas import tpu as pltpu
from jax.experimental.pallas import tpu_sc as plsc

_BATCH = 4096
_SEQ = 200
_EMBED = 64
_N = _BATCH * _SEQ

_WINDOW = 128

_MESH = plsc.VectorSubcoreMesh(core_axis_name="core", subcore_axis_name="subcore")


@jax.jit
def kernel(input_ids, weight):
    idx = input_ids.reshape(1, _N)

    @pl.kernel(
        out_type=jax.ShapeDtypeStruct((_N, _EMBED), jnp.float32),
        mesh=_MESH,
        compiler_params=pltpu.CompilerParams(use_tc_tiling_on_sc=False),
    )
    def gather_kernel(w_hbm, i_hbm, o_hbm):
        def body(i_vmem, o_vmem):
            pltpu.sync_copy(w_hbm.at[i_vmem.at[0]], o_vmem)

        pltpu.emit_pipeline(
            body,
            grid=(_N // _WINDOW,),
            in_specs=[pl.BlockSpec((1, _WINDOW), lambda i: (0, i))],
            out_specs=[pl.BlockSpec((_WINDOW, _EMBED), lambda i: (i, 0))],
            core_axis_name=("core", "subcore"),
            dimension_semantics=(pltpu.PARALLEL,),
        )(i_hbm, o_hbm)

    out = gather_kernel(weight, idx)
    return out.reshape(_BATCH, _SEQ, _EMBED)


# speedup vs baseline: 1.0695x; 1.0695x over previous
import jax
import jax.numpy as jnp
from jax import lax
from jax.experimental import pallas as pl
from jax.experimental.pallas import tpu as pltpu
from jax.experimental.pallas import tpu_sc as plsc

_BATCH = 4096
_SEQ = 200
_EMBED = 64
_N = _BATCH * _SEQ
_NW = 32
_PER_W = _N // _NW
_CH = 512
_GW = 512
_NCH = _PER_W // _CH

_MESH = plsc.VectorSubcoreMesh(core_axis_name="core", subcore_axis_name="subcore")


@jax.jit
def kernel(input_ids, weight):
    idx = input_ids.reshape(_N)

    @pl.kernel(
        out_type=jax.ShapeDtypeStruct((_N, _EMBED), jnp.float32),
        mesh=_MESH,
        scratch_types=[
            pltpu.VMEM((_PER_W,), jnp.int32),
            pltpu.VMEM((_CH, _EMBED), jnp.float32),
            pltpu.VMEM((_CH, _EMBED), jnp.float32),
            pltpu.SemaphoreType.DMA,
            pltpu.SemaphoreType.DMA,
            pltpu.SemaphoreType.DMA,
            pltpu.SemaphoreType.DMA,
        ],
        compiler_params=pltpu.CompilerParams(use_tc_tiling_on_sc=False),
    )
    def gather_kernel(w_hbm, i_hbm, o_hbm, idx_v, buf0, buf1, g0, g1, w0, w1):
        wid = lax.axis_index("subcore") * 2 + lax.axis_index("core")
        base = wid * _PER_W
        pltpu.sync_copy(i_hbm.at[pl.ds(base, _PER_W)], idx_v)

        def fire_gather(c, buf, sem):
            for j in range(_CH // _GW):
                pltpu.async_copy(
                    w_hbm.at[idx_v.at[pl.ds(c * _CH + j * _GW, _GW)]],
                    buf.at[pl.ds(j * _GW, _GW)],
                    sem,
                )

        def drain_gather(buf, sem):
            pltpu.make_async_copy(w_hbm.at[pl.ds(0, _CH)], buf, sem).wait()

        fire_gather(0, buf0, g0)
        fire_gather(1, buf1, g1)

        @pl.loop(0, _NCH, step=2)
        def _(c):
            drain_gather(buf0, g0)
            wb0 = pltpu.async_copy(buf0, o_hbm.at[pl.ds(base + c * _CH, _CH)], w0)
            drain_gather(buf1, g1)
            wb1 = pltpu.async_copy(
                buf1, o_hbm.at[pl.ds(base + (c + 1) * _CH, _CH)], w1
            )
            wb0.wait()

            @pl.when(c + 2 < _NCH)
            def _():
                fire_gather(c + 2, buf0, g0)

            wb1.wait()

            @pl.when(c + 3 < _NCH)
            def _():
                fire_gather(c + 3, buf1, g1)

    out = gather_kernel(weight, idx)
    return out.reshape(_BATCH, _SEQ, _EMBED)


# speedup vs baseline: 1.5231x; 1.4242x over previous
import jax
import jax.numpy as jnp
from jax import lax
from jax.experimental import pallas as pl
from jax.experimental.pallas import tpu as pltpu
from jax.experimental.pallas import tpu_sc as plsc

_BATCH = 4096
_SEQ = 200
_EMBED = 64
_N = _BATCH * _SEQ
_NW = 32
_PER_W = _N // _NW
_CH = 512
_NCH = _PER_W // _CH

_MESH = plsc.VectorSubcoreMesh(core_axis_name="core", subcore_axis_name="subcore")


@jax.jit
def kernel(input_ids, weight):
    idx = input_ids.reshape(_N) * 2
    w128 = jnp.pad(weight, ((0, 0), (0, _EMBED)))
    w2m = w128.reshape(2 * len(weight), _EMBED)

    @pl.kernel(
        out_type=jax.ShapeDtypeStruct((_N, 2 * _EMBED), jnp.float32),
        mesh=_MESH,
        scratch_types=[
            pltpu.VMEM((_PER_W,), jnp.int32),
            pltpu.VMEM((_CH, _EMBED), jnp.float32),
            pltpu.VMEM((_CH, _EMBED), jnp.float32),
            pltpu.SemaphoreType.DMA,
            pltpu.SemaphoreType.DMA,
            pltpu.SemaphoreType.DMA,
            pltpu.SemaphoreType.DMA,
        ],
        compiler_params=pltpu.CompilerParams(use_tc_tiling_on_sc=False),
    )
    def gather_kernel(w_hbm, i_hbm, o_hbm, idx_v, buf0, buf1, g0, g1, w0, w1):
        wid = lax.axis_index("subcore") * 2 + lax.axis_index("core")
        base = wid * _PER_W
        pltpu.sync_copy(i_hbm.at[pl.ds(base, _PER_W)], idx_v)

        def fire_gather(c, buf, sem):
            pltpu.async_copy(
                w_hbm.at[idx_v.at[pl.ds(c * _CH, _CH)]], buf, sem
            )

        def drain_gather(buf, sem):
            pltpu.make_async_copy(w_hbm.at[pl.ds(0, _CH)], buf, sem).wait()

        fire_gather(0, buf0, g0)
        fire_gather(1, buf1, g1)

        @pl.loop(0, _NCH, step=2)
        def _(c):
            drain_gather(buf0, g0)
            wb0 = pltpu.async_copy(
                buf0,
                o_hbm.at[pl.ds(base + c * _CH, _CH), pl.ds(0, _EMBED)],
                w0,
            )
            drain_gather(buf1, g1)
            wb1 = pltpu.async_copy(
                buf1,
                o_hbm.at[pl.ds(base + (c + 1) * _CH, _CH), pl.ds(0, _EMBED)],
                w1,
            )
            wb0.wait()

            @pl.when(c + 2 < _NCH)
            def _():
                fire_gather(c + 2, buf0, g0)

            wb1.wait()

            @pl.when(c + 3 < _NCH)
            def _():
                fire_gather(c + 3, buf1, g1)

    out = gather_kernel(w2m, idx)
    return out[:, :_EMBED].reshape(_BATCH, _SEQ, _EMBED)


# speedup vs baseline: 1.9693x; 1.2929x over previous
import jax
import jax.numpy as jnp
from jax import lax
from jax.experimental import pallas as pl
from jax.experimental.pallas import tpu as pltpu
from jax.experimental.pallas import tpu_sc as plsc

_BATCH = 4096
_SEQ = 200
_EMBED = 64
_VOCAB = 1000000
_N = _BATCH * _SEQ
_NW = 32
_PER_W = _N // _NW
_CH = 512
_NCH = _PER_W // _CH
_BK = 8192

_MESH = plsc.VectorSubcoreMesh(core_axis_name="core", subcore_axis_name="subcore")


def _repack_table(wt):
    def body(i_ref, o_ref):
        t = i_ref[...].T
        o_ref[:, 0:_EMBED] = t
        o_ref[:, _EMBED:] = t

    grid = (_VOCAB + _BK - 1) // _BK
    return pl.pallas_call(
        body,
        grid=(grid,),
        in_specs=[pl.BlockSpec((_EMBED, _BK), lambda i: (0, i))],
        out_specs=pl.BlockSpec((_BK, 2 * _EMBED), lambda i: (i, 0)),
        out_shape=jax.ShapeDtypeStruct((_VOCAB, 2 * _EMBED), jnp.float32),
    )(wt)


@jax.jit
def kernel(input_ids, weight):
    idx = input_ids.reshape(_N) * 2
    w128 = _repack_table(weight.T)
    w2m = w128.reshape(2 * _VOCAB, _EMBED)

    @pl.kernel(
        out_type=jax.ShapeDtypeStruct((_N, 2 * _EMBED), jnp.float32),
        mesh=_MESH,
        scratch_types=[
            pltpu.VMEM((_PER_W,), jnp.int32),
            pltpu.VMEM((_CH, _EMBED), jnp.float32),
            pltpu.VMEM((_CH, _EMBED), jnp.float32),
            pltpu.SemaphoreType.DMA,
            pltpu.SemaphoreType.DMA,
            pltpu.SemaphoreType.DMA,
            pltpu.SemaphoreType.DMA,
        ],
        compiler_params=pltpu.CompilerParams(use_tc_tiling_on_sc=False),
    )
    def gather_kernel(w_hbm, i_hbm, o_hbm, idx_v, buf0, buf1, g0, g1, w0, w1):
        wid = lax.axis_index("subcore") * 2 + lax.axis_index("core")
        base = wid * _PER_W
        pltpu.sync_copy(i_hbm.at[pl.ds(base, _PER_W)], idx_v)

        def fire_gather(c, buf, sem):
            pltpu.async_copy(
                w_hbm.at[idx_v.at[pl.ds(c * _CH, _CH)]], buf, sem
            )

        def drain_gather(buf, sem):
            pltpu.make_async_copy(w_hbm.at[pl.ds(0, _CH)], buf, sem).wait()

        fire_gather(0, buf0, g0)
        fire_gather(1, buf1, g1)

        @pl.loop(0, _NCH, step=2)
        def _(c):
            drain_gather(buf0, g0)
            wb0 = pltpu.async_copy(
                buf0,
                o_hbm.at[pl.ds(base + c * _CH, _CH), pl.ds(0, _EMBED)],
                w0,
            )
            drain_gather(buf1, g1)
            wb1 = pltpu.async_copy(
                buf1,
                o_hbm.at[pl.ds(base + (c + 1) * _CH, _CH), pl.ds(0, _EMBED)],
                w1,
            )
            wb0.wait()

            @pl.when(c + 2 < _NCH)
            def _():
                fire_gather(c + 2, buf0, g0)

            wb1.wait()

            @pl.when(c + 3 < _NCH)
            def _():
                fire_gather(c + 3, buf1, g1)

    out = gather_kernel(w2m, idx)
    return out[:, :_EMBED].reshape(_BATCH, _SEQ, _EMBED)


# speedup vs baseline: 2.0743x; 1.0533x over previous
import jax
import jax.numpy as jnp
from jax import lax
from jax.experimental import pallas as pl
from jax.experimental.pallas import tpu as pltpu
from jax.experimental.pallas import tpu_sc as plsc

_BATCH = 4096
_SEQ = 200
_EMBED = 64
_VOCAB = 1000000
_N = _BATCH * _SEQ
_NW = 32
_PER_W = _N // _NW
_CH = 512
_NCH = _PER_W // _CH
_BK = 16384

_MESH = plsc.VectorSubcoreMesh(core_axis_name="core", subcore_axis_name="subcore")


def _repack_table(wt):
    def body(i_ref, o_ref):
        t = i_ref[...].T
        o_ref[:, 0:_EMBED] = t
        o_ref[:, _EMBED:] = t

    grid = (_VOCAB + _BK - 1) // _BK
    return pl.pallas_call(
        body,
        grid=(grid,),
        in_specs=[pl.BlockSpec((_EMBED, _BK), lambda i: (0, i))],
        out_specs=pl.BlockSpec((_BK, 2 * _EMBED), lambda i: (i, 0)),
        out_shape=jax.ShapeDtypeStruct((_VOCAB, 2 * _EMBED), jnp.float32),
    )(wt)


@jax.jit
def kernel(input_ids, weight):
    idx = input_ids.reshape(_N) * 2
    w128 = _repack_table(weight.T)
    w2m = w128.reshape(2 * _VOCAB, _EMBED)

    @pl.kernel(
        out_type=jax.ShapeDtypeStruct((_N, 2 * _EMBED), jnp.float32),
        mesh=_MESH,
        scratch_types=[
            pltpu.VMEM((_PER_W,), jnp.int32),
            pltpu.VMEM((_CH, _EMBED), jnp.float32),
            pltpu.VMEM((_CH, _EMBED), jnp.float32),
            pltpu.SemaphoreType.DMA,
            pltpu.SemaphoreType.DMA,
            pltpu.SemaphoreType.DMA,
            pltpu.SemaphoreType.DMA,
        ],
        compiler_params=pltpu.CompilerParams(use_tc_tiling_on_sc=False),
    )
    def gather_kernel(w_hbm, i_hbm, o_hbm, idx_v, buf0, buf1, g0, g1, w0, w1):
        wid = lax.axis_index("subcore") * 2 + lax.axis_index("core")
        base = wid * _PER_W
        pltpu.sync_copy(i_hbm.at[pl.ds(base, _PER_W)], idx_v)

        def fire_gather(c, buf, sem):
            pltpu.async_copy(
                w_hbm.at[idx_v.at[pl.ds(c * _CH, _CH)]], buf, sem
            )

        def drain_gather(buf, sem):
            pltpu.make_async_copy(w_hbm.at[pl.ds(0, _CH)], buf, sem).wait()

        fire_gather(0, buf0, g0)
        fire_gather(1, buf1, g1)

        @pl.loop(0, _NCH, step=2)
        def _(c):
            drain_gather(buf0, g0)
            wb0 = pltpu.async_copy(
                buf0,
                o_hbm.at[pl.ds(base + c * _CH, _CH), pl.ds(0, _EMBED)],
                w0,
            )
            drain_gather(buf1, g1)
            wb1 = pltpu.async_copy(
                buf1,
                o_hbm.at[pl.ds(base + (c + 1) * _CH, _CH), pl.ds(0, _EMBED)],
                w1,
            )
            wb0.wait()

            @pl.when(c + 2 < _NCH)
            def _():
                fire_gather(c + 2, buf0, g0)

            wb1.wait()

            @pl.when(c + 3 < _NCH)
            def _():
                fire_gather(c + 3, buf1, g1)

    out = gather_kernel(w2m, idx)
    return out[:, :_EMBED].reshape(_BATCH, _SEQ, _EMBED)


# speedup vs baseline: 2.1100x; 1.0172x over previous
import jax
import jax.numpy as jnp
from jax import lax
from jax.experimental import pallas as pl
from jax.experimental.pallas import tpu as pltpu
from jax.experimental.pallas import tpu_sc as plsc

_BATCH = 4096
_SEQ = 200
_EMBED = 64
_VOCAB = 1000000
_N = _BATCH * _SEQ
_NW = 32
_PER_W = _N // _NW
_CH = 512
_NCH = _PER_W // _CH
_BK = 24576

_MESH = plsc.VectorSubcoreMesh(core_axis_name="core", subcore_axis_name="subcore")


def _repack_table(wt):
    def body(i_ref, o_ref):
        t = i_ref[...].T
        o_ref[:, 0:_EMBED] = t
        o_ref[:, _EMBED:] = t

    grid = (_VOCAB + _BK - 1) // _BK
    return pl.pallas_call(
        body,
        grid=(grid,),
        in_specs=[pl.BlockSpec((_EMBED, _BK), lambda i: (0, i))],
        out_specs=pl.BlockSpec((_BK, 2 * _EMBED), lambda i: (i, 0)),
        out_shape=jax.ShapeDtypeStruct((_VOCAB, 2 * _EMBED), jnp.float32),
    )(wt)


@jax.jit
def kernel(input_ids, weight):
    idx = input_ids.reshape(_N) * 2
    w128 = _repack_table(weight.T)
    w2m = w128.reshape(2 * _VOCAB, _EMBED)

    @pl.kernel(
        out_type=jax.ShapeDtypeStruct((_N, 2 * _EMBED), jnp.float32),
        mesh=_MESH,
        scratch_types=[
            pltpu.VMEM((_PER_W,), jnp.int32),
            pltpu.VMEM((_CH, _EMBED), jnp.float32),
            pltpu.VMEM((_CH, _EMBED), jnp.float32),
            pltpu.SemaphoreType.DMA,
            pltpu.SemaphoreType.DMA,
            pltpu.SemaphoreType.DMA,
            pltpu.SemaphoreType.DMA,
        ],
        compiler_params=pltpu.CompilerParams(use_tc_tiling_on_sc=False),
    )
    def gather_kernel(w_hbm, i_hbm, o_hbm, idx_v, buf0, buf1, g0, g1, w0, w1):
        wid = lax.axis_index("subcore") * 2 + lax.axis_index("core")
        base = wid * _PER_W
        pltpu.sync_copy(i_hbm.at[pl.ds(base, _PER_W)], idx_v)

        def fire_gather(c, buf, sem):
            pltpu.async_copy(
                w_hbm.at[idx_v.at[pl.ds(c * _CH, _CH)]], buf, sem
            )

        def drain_gather(buf, sem):
            pltpu.make_async_copy(w_hbm.at[pl.ds(0, _CH)], buf, sem).wait()

        fire_gather(0, buf0, g0)
        fire_gather(1, buf1, g1)

        @pl.loop(0, _NCH, step=2)
        def _(c):
            drain_gather(buf0, g0)
            wb0 = pltpu.async_copy(
                buf0,
                o_hbm.at[pl.ds(base + c * _CH, _CH), pl.ds(0, _EMBED)],
                w0,
            )
            drain_gather(buf1, g1)
            wb1 = pltpu.async_copy(
                buf1,
                o_hbm.at[pl.ds(base + (c + 1) * _CH, _CH), pl.ds(0, _EMBED)],
                w1,
            )
            wb0.wait()

            @pl.when(c + 2 < _NCH)
            def _():
                fire_gather(c + 2, buf0, g0)

            wb1.wait()

            @pl.when(c + 3 < _NCH)
            def _():
                fire_gather(c + 3, buf1, g1)

    out = gather_kernel(w2m, idx)
    return out[:, :_EMBED].reshape(_BATCH, _SEQ, _EMBED)
